# baseline (speedup 1.0000x reference)
import jax
import jax.numpy as jnp
from jax import lax
from jax.experimental import pallas as pl
from jax.experimental.pallas import tpu as pltpu
from jax.experimental.pallas import tpu_sc as plsc

NC = 2
NS = 16
NW = NC * NS
W = 128
KJ = 16
CH = 4
R = 8
LN2 = 0.6931471805599453

_L2C = (0.0439286278, -0.409475586, 1.61017755, -3.52021884, 5.06975632,
        -2.79415368)

_SC_PARAMS = pltpu.CompilerParams(use_tc_tiling_on_sc=False)


def _sc_pass(xt8, srcw, dstw, zeros, e4, *, np_, nwin, with_deg):
    slab = np_ // NS
    base = nwin // NW
    rem = nwin % NW
    mesh = plsc.VectorSubcoreMesh(core_axis_name="c", subcore_axis_name="s")
    out_type = jax.ShapeDtypeStruct((NC, np_, R), jnp.float32)
    scratch = [pltpu.VMEM_SHARED((np_, R), jnp.float32),
               pltpu.VMEM((KJ, W), jnp.int32),
               pltpu.VMEM((KJ, W), jnp.int32),
               pltpu.VMEM((KJ * W, R), jnp.float32),
               pltpu.VMEM((W, R), jnp.float32),
               pltpu.SemaphoreType.DMA,
               pltpu.SemaphoreType.DMA]

    def body(xt_h, srcw_h, dstw_h, zeros_h, e4_h, s_out, acc,
             srcb, dstb, msg, e4_v, sem_g, sem_s):
        cid = lax.axis_index("c")
        sid = lax.axis_index("s")
        row0 = sid * slab
        pltpu.sync_copy(zeros_h, acc.at[pl.ds(row0, slab)])
        if with_deg:
            pltpu.sync_copy(e4_h, e4_v)
        plsc.subcore_barrier()

        wid = cid * NS + sid
        wstart = wid * base + jnp.minimum(wid, rem)
        wcnt = base + jnp.where(wid < rem, 1, 0)
        nfull = wcnt // KJ

        def do_windows(nj):
            cps = []
            for j in range(nj):
                cps.append(pltpu.async_copy(
                    xt_h.at[srcb.at[j]], msg.at[pl.ds(j * W, W)], sem_g))
            for c in cps:
                c.wait()
            cps = []
            for j in range(nj):
                cps.append(pltpu.async_copy(
                    msg.at[pl.ds(j * W, W)], acc.at[dstb.at[j]], sem_s,
                    add=True))
                if with_deg:
                    cps.append(pltpu.async_copy(
                        e4_v, acc.at[srcb.at[j]], sem_s, add=True))
            for c in cps:
                c.wait()

        def step(g, carry):
            row = wstart + g * KJ
            pltpu.sync_copy(srcw_h.at[pl.ds(row, KJ)], srcb)
            pltpu.sync_copy(dstw_h.at[pl.ds(row, KJ)], dstb)
            do_windows(KJ)
            return carry

        lax.fori_loop(0, nfull, step, 0)

        def tail_step(t, carry):
            row = wstart + nfull * KJ + t
            pltpu.sync_copy(srcw_h.at[pl.ds(row, 1)], srcb.at[pl.ds(0, 1)])
            pltpu.sync_copy(dstw_h.at[pl.ds(row, 1)], dstb.at[pl.ds(0, 1)])
            do_windows(1)
            return carry

        lax.fori_loop(0, wcnt - nfull * KJ, tail_step, 0)

        plsc.subcore_barrier()
        pltpu.sync_copy(acc.at[pl.ds(row0, slab)],
                        s_out.at[cid, pl.ds(row0, slab)])

    f = pl.kernel(body, out_type=out_type, mesh=mesh,
                  scratch_types=scratch, compiler_params=_SC_PARAMS)
    return f(xt8, srcw, dstw, zeros, e4)


def _sc_combine(x16, sp16, dg16, scal16, *, deg_from_s):
    q8 = x16.shape[0]
    share = q8 // NW
    ck = share // 2
    mesh = plsc.VectorSubcoreMesh(core_axis_name="c", subcore_axis_name="s")
    bufs = 4 if deg_from_s else 6
    scratch = ([pltpu.VMEM((ck, 16), jnp.float32)] * bufs
               + [pltpu.VMEM((8, 16), jnp.float32)])

    def body(x_h, sp_h, dg_h, scal_h, o_h, *rest):
        if deg_from_s:
            x_v, sa_v, sb_v, o_v, sc_v = rest
            da_v = db_v = None
        else:
            x_v, sa_v, sb_v, da_v, db_v, o_v, sc_v = rest
        cid = lax.axis_index("c")
        sid = lax.axis_index("s")
        row0 = (cid * NS + sid) * share
        pltpu.sync_copy(scal_h, sc_v)
        k1 = sc_v[0]
        k2 = sc_v[1]
        sw = sc_v[2]
        nw = sc_v[3]
        b = sc_v[4]
        iota = lax.iota(jnp.int32, 16)
        bidx = (iota & 8) + 4

        for k in range(2):
            base = row0 + k * ck
            pltpu.sync_copy(x_h.at[pl.ds(base, ck)], x_v)
            pltpu.sync_copy(sp_h.at[0, pl.ds(base, ck)], sa_v)
            pltpu.sync_copy(sp_h.at[1, pl.ds(base, ck)], sb_v)
            if not deg_from_s:
                pltpu.sync_copy(dg_h.at[0, pl.ds(base, ck)], da_v)
                pltpu.sync_copy(dg_h.at[1, pl.ds(base, ck)], db_v)

            def step(i, carry):
                s = sa_v[i] + sb_v[i]
                dsrc = s if deg_from_s else da_v[i] + db_v[i]
                deg = jnp.take(dsrc, bidx)
                bits = lax.bitcast_convert_type(deg, jnp.int32)
                e = (bits >> 23) - 127
                mant = lax.bitcast_convert_type(
                    (bits & 0x007FFFFF) | 0x3F800000, jnp.float32)
                p = jnp.full_like(mant, _L2C[0])
                for c in _L2C[1:]:
                    p = p * mant + c
                l2 = e.astype(jnp.float32) + p
                f1 = jnp.exp(k1 * l2)
                f2 = jnp.exp(k2 * l2)
                o_v[i] = sw * f1 * x_v[i] + nw * f2 * s + b
                return carry

            lax.fori_loop(0, ck, step, 0)
            pltpu.sync_copy(o_v, o_h.at[pl.ds(base, ck)])

    f = pl.kernel(body, out_type=jax.ShapeDtypeStruct((q8, 16), jnp.float32),
                  mesh=mesh, scratch_types=scratch, compiler_params=_SC_PARAMS)
    return f(x16, sp16, dg16, scal16)


def _sc_pack(xp):
    np_ = xp.shape[1]
    cnt = np_ // NW
    nblk = cnt // 16
    mesh = plsc.VectorSubcoreMesh(core_axis_name="c", subcore_axis_name="s")
    scratch = ([pltpu.VMEM((cnt,), jnp.float32)] * CH
               + [pltpu.VMEM((cnt // 2, 16), jnp.float32)])

    def body(x_h, o_h, c0, c1, c2, c3, o_v):
        xc = (c0, c1, c2, c3)
        cid = lax.axis_index("c")
        sid = lax.axis_index("s")
        wid = cid * NS + sid
        n0 = wid * cnt
        for c in range(CH):
            pltpu.sync_copy(x_h.at[c, pl.ds(n0, cnt)], xc[c])
        iota = lax.iota(jnp.int32, 16)
        half = iota >> 3
        lane = iota & 7

        def blk(i, carry):
            vs = [xc[c][pl.ds(i * 16, 16)] for c in range(CH)]
            for j in range(8):
                idx = 2 * j + half
                o = jnp.where(lane == 3, jnp.take(vs[3], idx), 0.0)
                for c in (2, 1, 0):
                    o = jnp.where(lane == c, jnp.take(vs[c], idx), o)
                o_v[i * 8 + j] = o
            return carry

        lax.fori_loop(0, nblk, blk, 0)
        pltpu.sync_copy(o_v, o_h.at[pl.ds(wid * (cnt // 2), cnt // 2)])

    f = pl.kernel(body,
                  out_type=jax.ShapeDtypeStruct((np_ // 2, 16), jnp.float32),
                  mesh=mesh, scratch_types=scratch,
                  compiler_params=_SC_PARAMS)
    return f(xp)


def _sc_unpack(o16):
    q8 = o16.shape[0]
    np_ = q8 * 2
    cnt = np_ // NW
    nblk = cnt // 16
    mesh = plsc.VectorSubcoreMesh(core_axis_name="c", subcore_axis_name="s")
    scratch = ([pltpu.VMEM((cnt // 2, 16), jnp.float32)]
               + [pltpu.VMEM((cnt,), jnp.float32)] * CH)

    def body(o_h, y_h, o_v, c0, c1, c2, c3):
        yc = (c0, c1, c2, c3)
        cid = lax.axis_index("c")
        sid = lax.axis_index("s")
        wid = cid * NS + sid
        n0 = wid * cnt
        pltpu.sync_copy(o_h.at[pl.ds(wid * (cnt // 2), cnt // 2)], o_v)
        iota = lax.iota(jnp.int32, 16)
        pair = iota >> 1
        odd = iota & 1

        def blk(i, carry):
            vs = [o_v[i * 8 + j] for j in range(8)]
            for c in range(CH):
                idx = c + 8 * odd
                v = jnp.where(pair == 7, jnp.take(vs[7], idx), 0.0)
                for j in range(6, -1, -1):
                    v = jnp.where(pair == j, jnp.take(vs[j], idx), v)
                yc[c][pl.ds(i * 16, 16)] = v
            return carry

        lax.fori_loop(0, nblk, blk, 0)
        for c in range(CH):
            pltpu.sync_copy(yc[c], y_h.at[c, pl.ds(n0, cnt)])

    f = pl.kernel(body,
                  out_type=jax.ShapeDtypeStruct((CH, np_), jnp.float32),
                  mesh=mesh, scratch_types=scratch,
                  compiler_params=_SC_PARAMS)
    return f(o16)


def kernel(x, edge_index, alpha1, alpha2, gamma, bias):
    t_ch, n = x.shape
    e = edge_index.shape[1]
    n_layers = alpha1.shape[0]
    np_ = (n // 2048 + 1) * 2048
    q8 = np_ // 2
    nwin = e // W

    srcw = edge_index[0].reshape(nwin, W)
    dstw = edge_index[1].reshape(nwin, W)
    xp = jnp.pad(x, ((0, 0), (0, np_ - n)))
    zeros = jnp.zeros((np_ // NS, R), jnp.float32)
    e4 = jnp.zeros((W, R), jnp.float32).at[:, CH].set(1.0)

    out16 = _sc_pack(xp)
    dg16 = None
    for i in range(n_layers):
        sp = _sc_pass(out16.reshape(np_, R), srcw, dstw, zeros, e4,
                      np_=np_, nwin=nwin, with_deg=(i == 0))
        sp16 = sp.reshape(NC, q8, 16)
        if i == 0:
            dg16 = sp16
        a1 = alpha1[i, 0, 0]
        dp = jax.nn.sigmoid(gamma[i, 0, 0])
        sw = jnp.exp(a1)
        scal = jnp.stack([dp * LN2, (dp - 1.0) * LN2, sw, sw * jnp.tanh(a1),
                          bias[i, 0, 0], 0.0, 0.0, 0.0])
        scal16 = jnp.broadcast_to(scal[:, None], (8, 16))
        out16 = _sc_combine(out16, sp16, dg16, scal16, deg_from_s=(i == 0))
    return _sc_unpack(out16)[:, :n]


# speedup vs baseline: 1.4195x; 1.4195x over previous
import jax
import jax.numpy as jnp
from jax import lax
from jax.experimental import pallas as pl
from jax.experimental.pallas import tpu as pltpu
from jax.experimental.pallas import tpu_sc as plsc

NC = 2
NS = 16
NW = NC * NS
W = 128
KJ = 16
CH = 4
R = 8
LN2 = 0.6931471805599453

_L2C = (0.0439286278, -0.409475586, 1.61017755, -3.52021884, 5.06975632,
        -2.79415368)

_SC_PARAMS = pltpu.CompilerParams(use_tc_tiling_on_sc=False)


def _sc_pass(xt8, srcw, dstw, zeros, e4, *, np_, nwin, with_deg):
    slab = np_ // NS
    base = nwin // NW
    rem = nwin % NW
    mesh = plsc.VectorSubcoreMesh(core_axis_name="c", subcore_axis_name="s")
    out_type = jax.ShapeDtypeStruct((NC, np_, R), jnp.float32)
    scratch = [pltpu.VMEM_SHARED((np_, R), jnp.float32),
               pltpu.VMEM_SHARED((np_, R), jnp.float32),
               pltpu.VMEM((KJ, W), jnp.int32),
               pltpu.VMEM((KJ, W), jnp.int32),
               pltpu.VMEM((KJ * W, R), jnp.float32),
               pltpu.VMEM((W, R), jnp.float32),
               pltpu.SemaphoreType.DMA,
               pltpu.SemaphoreType.DMA]

    def body(xt_h, srcw_h, dstw_h, zeros_h, e4_h, s_out, tab, acc,
             srcb, dstb, msg, e4_v, sem_g, sem_s):
        cid = lax.axis_index("c")
        sid = lax.axis_index("s")
        row0 = sid * slab
        pltpu.sync_copy(zeros_h, acc.at[pl.ds(row0, slab)])
        if with_deg:
            pltpu.sync_copy(e4_h, e4_v)
        pltpu.sync_copy(xt_h.at[pl.ds(row0, slab)], tab.at[pl.ds(row0, slab)])
        plsc.subcore_barrier()

        wid = cid * NS + sid
        wstart = wid * base + jnp.minimum(wid, rem)
        wcnt = base + jnp.where(wid < rem, 1, 0)
        nfull = wcnt // KJ

        def do_windows(nj):
            cps = []
            for j in range(nj):
                cps.append(pltpu.async_copy(
                    tab.at[srcb.at[j]], msg.at[pl.ds(j * W, W)], sem_g))
            for c in cps:
                c.wait()
            cps = []
            for j in range(nj):
                cps.append(pltpu.async_copy(
                    msg.at[pl.ds(j * W, W)], acc.at[dstb.at[j]], sem_s,
                    add=True))
                if with_deg:
                    cps.append(pltpu.async_copy(
                        e4_v, acc.at[srcb.at[j]], sem_s, add=True))
            for c in cps:
                c.wait()

        def step(g, carry):
            row = wstart + g * KJ
            pltpu.sync_copy(srcw_h.at[pl.ds(row, KJ)], srcb)
            pltpu.sync_copy(dstw_h.at[pl.ds(row, KJ)], dstb)
            do_windows(KJ)
            return carry

        lax.fori_loop(0, nfull, step, 0)

        def tail_step(t, carry):
            row = wstart + nfull * KJ + t
            pltpu.sync_copy(srcw_h.at[pl.ds(row, 1)], srcb.at[pl.ds(0, 1)])
            pltpu.sync_copy(dstw_h.at[pl.ds(row, 1)], dstb.at[pl.ds(0, 1)])
            do_windows(1)
            return carry

        lax.fori_loop(0, wcnt - nfull * KJ, tail_step, 0)

        plsc.subcore_barrier()
        pltpu.sync_copy(acc.at[pl.ds(row0, slab)],
                        s_out.at[cid, pl.ds(row0, slab)])

    f = pl.kernel(body, out_type=out_type, mesh=mesh,
                  scratch_types=scratch, compiler_params=_SC_PARAMS)
    return f(xt8, srcw, dstw, zeros, e4)


def _sc_combine(x16, sp16, dg16, scal16, *, deg_from_s):
    q8 = x16.shape[0]
    share = q8 // NW
    ck = share // 2
    mesh = plsc.VectorSubcoreMesh(core_axis_name="c", subcore_axis_name="s")
    bufs = 4 if deg_from_s else 6
    scratch = ([pltpu.VMEM((ck, 16), jnp.float32)] * bufs
               + [pltpu.VMEM((8, 16), jnp.float32)])

    def body(x_h, sp_h, dg_h, scal_h, o_h, *rest):
        if deg_from_s:
            x_v, sa_v, sb_v, o_v, sc_v = rest
            da_v = db_v = None
        else:
            x_v, sa_v, sb_v, da_v, db_v, o_v, sc_v = rest
        cid = lax.axis_index("c")
        sid = lax.axis_index("s")
        row0 = (cid * NS + sid) * share
        pltpu.sync_copy(scal_h, sc_v)
        k1 = sc_v[0]
        k2 = sc_v[1]
        sw = sc_v[2]
        nw = sc_v[3]
        b = sc_v[4]
        iota = lax.iota(jnp.int32, 16)
        bidx = (iota & 8) + 4

        for k in range(2):
            base = row0 + k * ck
            pltpu.sync_copy(x_h.at[pl.ds(base, ck)], x_v)
            pltpu.sync_copy(sp_h.at[0, pl.ds(base, ck)], sa_v)
            pltpu.sync_copy(sp_h.at[1, pl.ds(base, ck)], sb_v)
            if not deg_from_s:
                pltpu.sync_copy(dg_h.at[0, pl.ds(base, ck)], da_v)
                pltpu.sync_copy(dg_h.at[1, pl.ds(base, ck)], db_v)

            def step(i, carry):
                s = sa_v[i] + sb_v[i]
                dsrc = s if deg_from_s else da_v[i] + db_v[i]
                deg = jnp.take(dsrc, bidx)
                bits = lax.bitcast_convert_type(deg, jnp.int32)
                e = (bits >> 23) - 127
                mant = lax.bitcast_convert_type(
                    (bits & 0x007FFFFF) | 0x3F800000, jnp.float32)
                p = jnp.full_like(mant, _L2C[0])
                for c in _L2C[1:]:
                    p = p * mant + c
                l2 = e.astype(jnp.float32) + p
                f1 = jnp.exp(k1 * l2)
                f2 = jnp.exp(k2 * l2)
                o_v[i] = sw * f1 * x_v[i] + nw * f2 * s + b
                return carry

            lax.fori_loop(0, ck, step, 0)
            pltpu.sync_copy(o_v, o_h.at[pl.ds(base, ck)])

    f = pl.kernel(body, out_type=jax.ShapeDtypeStruct((q8, 16), jnp.float32),
                  mesh=mesh, scratch_types=scratch, compiler_params=_SC_PARAMS)
    return f(x16, sp16, dg16, scal16)


def _sc_pack(xp):
    np_ = xp.shape[1]
    cnt = np_ // NW
    nblk = cnt // 16
    mesh = plsc.VectorSubcoreMesh(core_axis_name="c", subcore_axis_name="s")
    scratch = ([pltpu.VMEM((cnt,), jnp.float32)] * CH
               + [pltpu.VMEM((cnt // 2, 16), jnp.float32)])

    def body(x_h, o_h, c0, c1, c2, c3, o_v):
        xc = (c0, c1, c2, c3)
        cid = lax.axis_index("c")
        sid = lax.axis_index("s")
        wid = cid * NS + sid
        n0 = wid * cnt
        for c in range(CH):
            pltpu.sync_copy(x_h.at[c, pl.ds(n0, cnt)], xc[c])
        iota = lax.iota(jnp.int32, 16)
        half = iota >> 3
        lane = iota & 7

        def blk(i, carry):
            vs = [xc[c][pl.ds(i * 16, 16)] for c in range(CH)]
            for j in range(8):
                idx = 2 * j + half
                o = jnp.where(lane == 3, jnp.take(vs[3], idx), 0.0)
                for c in (2, 1, 0):
                    o = jnp.where(lane == c, jnp.take(vs[c], idx), o)
                o_v[i * 8 + j] = o
            return carry

        lax.fori_loop(0, nblk, blk, 0)
        pltpu.sync_copy(o_v, o_h.at[pl.ds(wid * (cnt // 2), cnt // 2)])

    f = pl.kernel(body,
                  out_type=jax.ShapeDtypeStruct((np_ // 2, 16), jnp.float32),
                  mesh=mesh, scratch_types=scratch,
                  compiler_params=_SC_PARAMS)
    return f(xp)


def _sc_unpack(o16):
    q8 = o16.shape[0]
    np_ = q8 * 2
    cnt = np_ // NW
    nblk = cnt // 16
    mesh = plsc.VectorSubcoreMesh(core_axis_name="c", subcore_axis_name="s")
    scratch = ([pltpu.VMEM((cnt // 2, 16), jnp.float32)]
               + [pltpu.VMEM((cnt,), jnp.float32)] * CH)

    def body(o_h, y_h, o_v, c0, c1, c2, c3):
        yc = (c0, c1, c2, c3)
        cid = lax.axis_index("c")
        sid = lax.axis_index("s")
        wid = cid * NS + sid
        n0 = wid * cnt
        pltpu.sync_copy(o_h.at[pl.ds(wid * (cnt // 2), cnt // 2)], o_v)
        iota = lax.iota(jnp.int32, 16)
        pair = iota >> 1
        odd = iota & 1

        def blk(i, carry):
            vs = [o_v[i * 8 + j] for j in range(8)]
            for c in range(CH):
                idx = c + 8 * odd
                v = jnp.where(pair == 7, jnp.take(vs[7], idx), 0.0)
                for j in range(6, -1, -1):
                    v = jnp.where(pair == j, jnp.take(vs[j], idx), v)
                yc[c][pl.ds(i * 16, 16)] = v
            return carry

        lax.fori_loop(0, nblk, blk, 0)
        for c in range(CH):
            pltpu.sync_copy(yc[c], y_h.at[c, pl.ds(n0, cnt)])

    f = pl.kernel(body,
                  out_type=jax.ShapeDtypeStruct((CH, np_), jnp.float32),
                  mesh=mesh, scratch_types=scratch,
                  compiler_params=_SC_PARAMS)
    return f(o16)


def kernel(x, edge_index, alpha1, alpha2, gamma, bias):
    t_ch, n = x.shape
    e = edge_index.shape[1]
    n_layers = alpha1.shape[0]
    np_ = (n // 2048 + 1) * 2048
    q8 = np_ // 2
    nwin = e // W

    srcw = edge_index[0].reshape(nwin, W)
    dstw = edge_index[1].reshape(nwin, W)
    xp = jnp.pad(x, ((0, 0), (0, np_ - n)))
    zeros = jnp.zeros((np_ // NS, R), jnp.float32)
    e4 = jnp.zeros((W, R), jnp.float32).at[:, CH].set(1.0)

    out16 = _sc_pack(xp)
    dg16 = None
    for i in range(n_layers):
        sp = _sc_pass(out16.reshape(np_, R), srcw, dstw, zeros, e4,
                      np_=np_, nwin=nwin, with_deg=(i == 0))
        sp16 = sp.reshape(NC, q8, 16)
        if i == 0:
            dg16 = sp16
        a1 = alpha1[i, 0, 0]
        dp = jax.nn.sigmoid(gamma[i, 0, 0])
        sw = jnp.exp(a1)
        scal = jnp.stack([dp * LN2, (dp - 1.0) * LN2, sw, sw * jnp.tanh(a1),
                          bias[i, 0, 0], 0.0, 0.0, 0.0])
        scal16 = jnp.broadcast_to(scal[:, None], (8, 16))
        out16 = _sc_combine(out16, sp16, dg16, scal16, deg_from_s=(i == 0))
    return _sc_unpack(out16)[:, :n]
